# baseline (speedup 1.0000x reference)
import functools

import jax
import jax.numpy as jnp
from jax import lax
from jax.experimental import pallas as pl
from jax.experimental.pallas import tpu as pltpu
from jax.experimental.pallas import tpu_sc as plsc

NC = 2
NS = 16
NW = NC * NS
CHUNK = 128
NBUF = 4
NSTREAM = 3


def _make_embed(N, V, D):
    BPW = N // NW
    NCHUNK = BPW // CHUNK
    NGRP = NCHUNK // NBUF
    QPC = CHUNK // 16
    mesh = plsc.VectorSubcoreMesh(
        core_axis_name="c", subcore_axis_name="s", num_cores=NC, num_subcores=NS
    )

    @functools.partial(
        pl.kernel,
        out_type=jax.ShapeDtypeStruct((N, D), jnp.float32),
        mesh=mesh,
        scratch_types=[
            pltpu.VMEM_SHARED((V, D), jnp.float32),
            pltpu.VMEM((V, D), jnp.float32),
            pltpu.VMEM((NCHUNK, CHUNK), jnp.int32),
            pltpu.VMEM((NBUF, CHUNK, D), jnp.float32),
        ]
        + [pltpu.SemaphoreType.DMA] * (NSTREAM + NBUF),
    )
    def embed(z_hbm, w_hbm, out_hbm, stable, table_v, idx_v, rows_v, *sems):
        gsems, wsems = sems[:NSTREAM], sems[NSTREAM:]
        sid = lax.axis_index("s")
        wid = sid * NC + lax.axis_index("c")
        base_chunk = wid * NCHUNK

        @pl.when(sid == 0)
        def _():
            pltpu.sync_copy(w_hbm, stable)

        pltpu.sync_copy(w_hbm, table_v)
        pltpu.sync_copy(z_hbm.at[pl.ds(base_chunk, NCHUNK)], idx_v)

        @plsc.parallel_loop(0, NCHUNK, 1)
        def suball(c):
            for k in range(CHUNK // 16):
                sl = pl.ds(k * 16, 16)
                idx_v[c, sl] = idx_v[c, sl] - 1

        plsc.subcore_barrier()

        def gather(c, b):
            return pltpu.make_async_copy(
                stable.at[idx_v.at[c]], rows_v.at[b], gsems[b]
            )

        def write(c, b):
            return pltpu.make_async_copy(
                rows_v.at[b],
                out_hbm.at[pl.ds((base_chunk + c) * CHUNK, CHUNK)],
                wsems[b],
            )

        def compute_chunk(c, b):
            @plsc.parallel_loop(0, QPC, 1)
            def row16(q):
                zv = idx_v[c, pl.ds(q * 16, 16)]
                for u in range(16):
                    zr = zv[u]
                    for k in range(D // 16):
                        sl = pl.ds(k * 16, 16)
                        rows_v[b, q * 16 + u, sl] = table_v[zr, sl]

        for b in range(NSTREAM):
            gather(b, b).start()

        def group(g, carry):
            base = g * NBUF

            @pl.when(g > 0)
            def _():
                write(base - 1, NSTREAM).wait()

            compute_chunk(base + NSTREAM, NSTREAM)
            write(base + NSTREAM, NSTREAM).start()

            for b in range(NSTREAM):
                gather(base + b, b).wait()
                write(base + b, b).start()
            for b in range(NSTREAM):
                c = base + b
                write(c, b).wait()

                @pl.when(c + NBUF < NCHUNK)
                def _():
                    gather(c + NBUF, b).start()

            return carry

        lax.fori_loop(0, NGRP, group, 0)
        write(NCHUNK - 1, NSTREAM).wait()

    return embed


def kernel(z, weight):
    (N,) = z.shape
    V, D = weight.shape
    z2 = z.reshape(N // CHUNK, CHUNK)
    return _make_embed(N, V, D)(z2, weight)


# speedup vs baseline: 1.0987x; 1.0987x over previous
import functools

import jax
import jax.numpy as jnp
from jax import lax
from jax.experimental import pallas as pl
from jax.experimental.pallas import tpu as pltpu
from jax.experimental.pallas import tpu_sc as plsc

NC = 2
NS = 16
NW = NC * NS
CHUNK = 128
NBUF = 4


def _make_embed(N, V, D):
    BPW = N // NW
    NCHUNK = BPW // CHUNK
    NGRP = NCHUNK // NBUF
    mesh = plsc.VectorSubcoreMesh(
        core_axis_name="c", subcore_axis_name="s", num_cores=NC, num_subcores=NS
    )

    @functools.partial(
        pl.kernel,
        out_type=jax.ShapeDtypeStruct((N, D), jnp.float32),
        mesh=mesh,
        scratch_types=[
            pltpu.VMEM_SHARED((V, D), jnp.float32),
            pltpu.VMEM((NCHUNK, CHUNK), jnp.int32),
            pltpu.VMEM((NBUF, CHUNK, D), jnp.float32),
        ]
        + [pltpu.SemaphoreType.DMA] * (2 * NBUF),
    )
    def embed(z_hbm, w_hbm, out_hbm, stable, idx_v, rows_v, *sems):
        gsems, wsems = sems[:NBUF], sems[NBUF:]
        sid = lax.axis_index("s")
        wid = sid * NC + lax.axis_index("c")
        base_chunk = wid * NCHUNK

        @pl.when(sid == 0)
        def _():
            pltpu.sync_copy(w_hbm, stable)

        pltpu.sync_copy(z_hbm.at[pl.ds(base_chunk, NCHUNK)], idx_v)

        @plsc.parallel_loop(0, NCHUNK, 1)
        def suball(c):
            for k in range(CHUNK // 16):
                sl = pl.ds(k * 16, 16)
                idx_v[c, sl] = idx_v[c, sl] - 1

        plsc.subcore_barrier()

        def gather(c, b):
            return pltpu.make_async_copy(
                stable.at[idx_v.at[c]], rows_v.at[b], gsems[b]
            )

        def write(c, b):
            return pltpu.make_async_copy(
                rows_v.at[b],
                out_hbm.at[pl.ds((base_chunk + c) * CHUNK, CHUNK)],
                wsems[b],
            )

        for b in range(NBUF):
            gather(b, b).start()

        def group(g, carry):
            base = g * NBUF
            for b in range(NBUF):
                gather(base + b, b).wait()
                write(base + b, b).start()
            for b in range(NBUF):
                c = base + b
                write(c, b).wait()

                @pl.when(c + NBUF < NCHUNK)
                def _():
                    gather(c + NBUF, b).start()

            return carry

        lax.fori_loop(0, NGRP, group, 0)

    return embed


def kernel(z, weight):
    (N,) = z.shape
    V, D = weight.shape
    z2 = z.reshape(N // CHUNK, CHUNK)
    return _make_embed(N, V, D)(z2, weight)


# speedup vs baseline: 1.1038x; 1.0047x over previous
import functools

import jax
import jax.numpy as jnp
from jax import lax
from jax.experimental import pallas as pl
from jax.experimental.pallas import tpu as pltpu
from jax.experimental.pallas import tpu_sc as plsc

NC = 2
NS = 16
NW = NC * NS
CHUNK = 128
NBUF = 4


def _make_embed(N, V, D):
    BPW = N // NW
    NCHUNK = BPW // CHUNK
    NGRP = NCHUNK // NBUF
    mesh = plsc.VectorSubcoreMesh(
        core_axis_name="c", subcore_axis_name="s", num_cores=NC, num_subcores=NS
    )

    @functools.partial(
        pl.kernel,
        out_type=jax.ShapeDtypeStruct((N, D), jnp.float32),
        mesh=mesh,
        scratch_types=[
            pltpu.VMEM_SHARED((V + 1, D), jnp.float32),
            pltpu.VMEM((NCHUNK, CHUNK), jnp.int32),
            pltpu.VMEM((NBUF, CHUNK, D), jnp.float32),
        ]
        + [pltpu.SemaphoreType.DMA] * (2 * NBUF),
    )
    def embed(z_hbm, w_hbm, out_hbm, stable, idx_v, rows_v, *sems):
        gsems, wsems = sems[:NBUF], sems[NBUF:]
        sid = lax.axis_index("s")
        wid = sid * NC + lax.axis_index("c")
        base_chunk = wid * NCHUNK

        @pl.when(sid == 0)
        def _():
            pltpu.sync_copy(w_hbm, stable.at[pl.ds(1, V)])

        pltpu.sync_copy(z_hbm.at[pl.ds(base_chunk, NCHUNK)], idx_v)

        plsc.subcore_barrier()

        def gather(c, b):
            return pltpu.make_async_copy(
                stable.at[idx_v.at[c]], rows_v.at[b], gsems[b]
            )

        def write(c, b):
            return pltpu.make_async_copy(
                rows_v.at[b],
                out_hbm.at[pl.ds((base_chunk + c) * CHUNK, CHUNK)],
                wsems[b],
            )

        for b in range(NBUF):
            gather(b, b).start()

        def group(g, carry):
            base = g * NBUF
            for b in range(NBUF):
                gather(base + b, b).wait()
                write(base + b, b).start()
            for b in range(NBUF):
                c = base + b
                write(c, b).wait()

                @pl.when(c + NBUF < NCHUNK)
                def _():
                    gather(c + NBUF, b).start()

            return carry

        lax.fori_loop(0, NGRP, group, 0)

    return embed


def kernel(z, weight):
    (N,) = z.shape
    V, D = weight.shape
    z2 = z.reshape(N // CHUNK, CHUNK)
    return _make_embed(N, V, D)(z2, weight)
